# Initial kernel scaffold; baseline (speedup 1.0000x reference)
import functools
import jax
import jax.numpy as jnp
from jax import lax
from jax.experimental import pallas as pl
from jax.experimental.pallas import tpu as pltpu
from jax.experimental.pallas import tpu_sc as plsc

N = 10000
E = 320000
D = 128

NC = 2
NS = 16
NW = NC * NS
K = 80
CHUNK_ROWS = E // K
W_CHUNKS = E // (NW * K)
NPW = N // NS
DW = 16

_MESH = plsc.VectorSubcoreMesh(core_axis_name="c", subcore_axis_name="s")


def _zero_shared(acc, zbuf, sub, width):
    zrows = zbuf.shape[0]

    def zero_row(r, carry):
        for cc in range(width // 16):
            zbuf[r, pl.ds(cc * 16, 16)] = jnp.zeros((16,), jnp.float32)
        return carry

    lax.fori_loop(0, zrows, zero_row, 0)
    for j in range(NPW // zrows):
        pltpu.sync_copy(zbuf, acc.at[pl.ds(sub * NPW + j * zrows, zrows)])


def _deg_body(dst_hbm, out_hbm, didx_v, ones_v, zbuf, acc):
    core = lax.axis_index("c")
    sub = lax.axis_index("s")
    wid = core * NS + sub

    def ones_row(r, carry):
        ones_v[r] = jnp.ones((16,), jnp.float32)
        return carry

    lax.fori_loop(0, K, ones_row, 0)
    _zero_shared(acc, zbuf, sub, DW)
    plsc.subcore_barrier()

    pltpu.sync_copy(dst_hbm.at[pl.ds(wid * W_CHUNKS, W_CHUNKS)], didx_v)

    def chunk(j, carry):
        pltpu.sync_copy(ones_v, acc.at[didx_v.at[j]], add=True)
        return carry

    lax.fori_loop(0, W_CHUNKS, chunk, 0)
    plsc.subcore_barrier()
    pltpu.sync_copy(acc.at[pl.ds(sub * NPW, NPW)],
                    out_hbm.at[core, pl.ds(sub * NPW, NPW)])


@functools.partial(
    pl.kernel,
    out_type=jax.ShapeDtypeStruct((NC, N, DW), jnp.float32),
    mesh=_MESH,
    scratch_types=[
        pltpu.VMEM((W_CHUNKS, K), jnp.int32),
        pltpu.VMEM((K, DW), jnp.float32),
        pltpu.VMEM((NPW, DW), jnp.float32),
        pltpu.VMEM_SHARED((N, DW), jnp.float32),
    ],
)
def _deg_kernel(dst_hbm, out_hbm, didx_v, ones_v, zbuf, acc):
    _deg_body(dst_hbm, out_hbm, didx_v, ones_v, zbuf, acc)


def _edge_body(g_hbm, src_hbm, dst_hbm, out_hbm,
               sidx_v, didx_v, rows_v, zbuf, acc, sem):
    core = lax.axis_index("c")
    sub = lax.axis_index("s")
    wid = core * NS + sub

    _zero_shared(acc, zbuf, sub, D)
    plsc.subcore_barrier()

    pltpu.sync_copy(src_hbm.at[pl.ds(wid * W_CHUNKS, W_CHUNKS)], sidx_v)
    pltpu.sync_copy(dst_hbm.at[pl.ds(wid * W_CHUNKS, W_CHUNKS)], didx_v)

    def chunk(j, carry):
        pltpu.async_copy(g_hbm.at[sidx_v.at[j]], rows_v, sem).wait()
        pltpu.sync_copy(rows_v, acc.at[didx_v.at[j]], add=True)
        return carry

    lax.fori_loop(0, W_CHUNKS, chunk, 0)
    plsc.subcore_barrier()
    pltpu.sync_copy(acc.at[pl.ds(sub * NPW, NPW)],
                    out_hbm.at[core, pl.ds(sub * NPW, NPW)])


@functools.partial(
    pl.kernel,
    out_type=jax.ShapeDtypeStruct((NC, N, D), jnp.float32),
    mesh=_MESH,
    scratch_types=[
        pltpu.VMEM((W_CHUNKS, K), jnp.int32),
        pltpu.VMEM((W_CHUNKS, K), jnp.int32),
        pltpu.VMEM((K, D), jnp.float32),
        pltpu.VMEM((NPW // 5, D), jnp.float32),
        pltpu.VMEM_SHARED((N, D), jnp.float32),
        pltpu.SemaphoreType.DMA,
    ],
)
def _edge_kernel(g_hbm, src_hbm, dst_hbm, out_hbm,
                 sidx_v, didx_v, rows_v, zbuf, acc, sem):
    _edge_body(g_hbm, src_hbm, dst_hbm, out_hbm,
               sidx_v, didx_v, rows_v, zbuf, acc, sem)


BR = 1000


def _tc1_body(d0_ref, d1_ref, x_ref, w1_ref, dinv_ref, g1_ref):
    deg = d0_ref[...] + d1_ref[...] + 1.0
    s = lax.rsqrt(jnp.maximum(deg, 1.0))
    bcast = jnp.full((DW, D), 1.0 / DW, jnp.float32)
    dinv = jnp.dot(s, bcast, preferred_element_type=jnp.float32)
    h = jnp.dot(x_ref[...], w1_ref[...], preferred_element_type=jnp.float32)
    dinv_ref[...] = dinv
    g1_ref[...] = h * dinv


def _tc_stage1(deg, x, w1):
    return pl.pallas_call(
        _tc1_body,
        grid=(N // BR,),
        in_specs=[
            pl.BlockSpec((BR, DW), lambda i: (i, 0)),
            pl.BlockSpec((BR, DW), lambda i: (i, 0)),
            pl.BlockSpec((BR, D), lambda i: (i, 0)),
            pl.BlockSpec((D, D), lambda i: (0, 0)),
        ],
        out_specs=[
            pl.BlockSpec((BR, D), lambda i: (i, 0)),
            pl.BlockSpec((BR, D), lambda i: (i, 0)),
        ],
        out_shape=[
            jax.ShapeDtypeStruct((N, D), jnp.float32),
            jax.ShapeDtypeStruct((N, D), jnp.float32),
        ],
    )(deg[0], deg[1], x, w1)


def _tc2_body(p0_ref, p1_ref, g_ref, dinv_ref, b_ref, w_ref, out_ref):
    agg = p0_ref[...] + p1_ref[...] + g_ref[...]
    xn = jax.nn.relu(agg * dinv_ref[...] + b_ref[...])
    h = jnp.dot(xn, w_ref[...], preferred_element_type=jnp.float32)
    out_ref[...] = h * dinv_ref[...]


def _tc3_body(p0_ref, p1_ref, g_ref, dinv_ref, b_ref, w_ref, b3_ref, out_ref):
    agg = p0_ref[...] + p1_ref[...] + g_ref[...]
    xn = jax.nn.relu(agg * dinv_ref[...] + b_ref[...])
    h = jnp.dot(xn, w_ref[...], preferred_element_type=jnp.float32)
    out_ref[...] = h + b3_ref[...]


def _tc_stage2(p, g, dinv, b, w):
    return pl.pallas_call(
        _tc2_body,
        grid=(N // BR,),
        in_specs=[
            pl.BlockSpec((BR, D), lambda i: (i, 0)),
            pl.BlockSpec((BR, D), lambda i: (i, 0)),
            pl.BlockSpec((BR, D), lambda i: (i, 0)),
            pl.BlockSpec((BR, D), lambda i: (i, 0)),
            pl.BlockSpec((1, D), lambda i: (0, 0)),
            pl.BlockSpec((D, D), lambda i: (0, 0)),
        ],
        out_specs=pl.BlockSpec((BR, D), lambda i: (i, 0)),
        out_shape=jax.ShapeDtypeStruct((N, D), jnp.float32),
    )(p[0], p[1], g, dinv, b.reshape(1, D), w)


def _tc_stage3(p, g, dinv, b, w, b3):
    return pl.pallas_call(
        _tc3_body,
        grid=(N // BR,),
        in_specs=[
            pl.BlockSpec((BR, D), lambda i: (i, 0)),
            pl.BlockSpec((BR, D), lambda i: (i, 0)),
            pl.BlockSpec((BR, D), lambda i: (i, 0)),
            pl.BlockSpec((BR, D), lambda i: (i, 0)),
            pl.BlockSpec((1, D), lambda i: (0, 0)),
            pl.BlockSpec((D, D), lambda i: (0, 0)),
            pl.BlockSpec((1, D), lambda i: (0, 0)),
        ],
        out_specs=pl.BlockSpec((BR, D), lambda i: (i, 0)),
        out_shape=jax.ShapeDtypeStruct((N, D), jnp.float32),
    )(p[0], p[1], g, dinv, b.reshape(1, D), w, b3.reshape(1, D))


@jax.jit
def kernel(x, edge_index, W1, b1, W2, b2, W3, b3):
    src = edge_index[0].reshape(CHUNK_ROWS, K)
    dst = edge_index[1].reshape(CHUNK_ROWS, K)

    deg = _deg_kernel(dst)
    dinv, g1 = _tc_stage1(deg, x, W1)
    p1 = _edge_kernel(g1, src, dst)
    g2 = _tc_stage2(p1, g1, dinv, b1, W2)
    p2 = _edge_kernel(g2, src, dst)
    return _tc_stage3(p2, g2, dinv, b2, W3, b3)


# speedup vs baseline: 17.3001x; 17.3001x over previous
import functools
import jax
import jax.numpy as jnp
from jax import lax
from jax.experimental import pallas as pl
from jax.experimental.pallas import tpu as pltpu
from jax.experimental.pallas import tpu_sc as plsc

N = 10000
E = 320000
D = 128

NC = 2
NS = 16
NW = NC * NS
K = 80
W_CHUNKS = E // (NW * K)
NPAD = 10240
NPW = NPAD // NS
ZR = 32

_MESH = plsc.VectorSubcoreMesh(core_axis_name="c", subcore_axis_name="s")


def _zero_shared(acc, zbuf, sub):
    def zero_row(r, carry):
        for cc in range(D // 16):
            zbuf[r, pl.ds(cc * 16, 16)] = jnp.zeros((16,), jnp.float32)
        return carry

    lax.fori_loop(0, ZR, zero_row, 0)
    for j in range(NPW // ZR):
        pltpu.sync_copy(zbuf, acc.at[pl.ds(sub * NPW + j * ZR, ZR)])


def _deg_body(dst_hbm, out_hbm, didx_v, ones_v, zbuf, acc):
    core = lax.axis_index("c")
    sub = lax.axis_index("s")
    wid = core * NS + sub

    def ones_row(r, carry):
        for cc in range(D // 16):
            ones_v[r, pl.ds(cc * 16, 16)] = jnp.ones((16,), jnp.float32)
        return carry

    lax.fori_loop(0, K, ones_row, 0)
    _zero_shared(acc, zbuf, sub)
    plsc.subcore_barrier()

    pltpu.sync_copy(dst_hbm.at[wid], didx_v)

    def chunk(j, carry):
        pltpu.sync_copy(ones_v, acc.at[didx_v.at[j]], add=True)
        return carry

    lax.fori_loop(0, W_CHUNKS, chunk, 0)
    plsc.subcore_barrier()
    pltpu.sync_copy(acc.at[pl.ds(sub * NPW, NPW)], out_hbm.at[core, sub])


@functools.partial(
    pl.kernel,
    out_type=jax.ShapeDtypeStruct((NC, NS, NPW, D), jnp.float32),
    mesh=_MESH,
    scratch_types=[
        pltpu.VMEM((W_CHUNKS, K), jnp.int32),
        pltpu.VMEM((K, D), jnp.float32),
        pltpu.VMEM((ZR, D), jnp.float32),
        pltpu.VMEM_SHARED((NPAD, D), jnp.float32),
    ],
)
def _deg_kernel(dst_hbm, out_hbm, didx_v, ones_v, zbuf, acc):
    _deg_body(dst_hbm, out_hbm, didx_v, ones_v, zbuf, acc)


def _edge_body(g_hbm, src_hbm, dst_hbm, out_hbm,
               sidx_v, didx_v, rows_v, zbuf, acc, sem):
    core = lax.axis_index("c")
    sub = lax.axis_index("s")
    wid = core * NS + sub

    _zero_shared(acc, zbuf, sub)
    plsc.subcore_barrier()

    pltpu.sync_copy(src_hbm.at[wid], sidx_v)
    pltpu.sync_copy(dst_hbm.at[wid], didx_v)

    def chunk(j, carry):
        pltpu.async_copy(g_hbm.at[sidx_v.at[j]], rows_v, sem).wait()
        pltpu.sync_copy(rows_v, acc.at[didx_v.at[j]], add=True)
        return carry

    lax.fori_loop(0, W_CHUNKS, chunk, 0)
    plsc.subcore_barrier()
    pltpu.sync_copy(acc.at[pl.ds(sub * NPW, NPW)], out_hbm.at[core, sub])


@functools.partial(
    pl.kernel,
    out_type=jax.ShapeDtypeStruct((NC, NS, NPW, D), jnp.float32),
    mesh=_MESH,
    scratch_types=[
        pltpu.VMEM((W_CHUNKS, K), jnp.int32),
        pltpu.VMEM((W_CHUNKS, K), jnp.int32),
        pltpu.VMEM((K, D), jnp.float32),
        pltpu.VMEM((ZR, D), jnp.float32),
        pltpu.VMEM_SHARED((NPAD, D), jnp.float32),
        pltpu.SemaphoreType.DMA,
    ],
)
def _edge_kernel(g_hbm, src_hbm, dst_hbm, out_hbm,
                 sidx_v, didx_v, rows_v, zbuf, acc, sem):
    _edge_body(g_hbm, src_hbm, dst_hbm, out_hbm,
               sidx_v, didx_v, rows_v, zbuf, acc, sem)


BR = 1000


def _tc1_body(d0_ref, d1_ref, x_ref, w1_ref, dinv_ref, g1_ref):
    deg = d0_ref[...] + d1_ref[...] + 1.0
    dinv = lax.rsqrt(jnp.maximum(deg, 1.0))
    h = jnp.dot(x_ref[...], w1_ref[...], preferred_element_type=jnp.float32)
    dinv_ref[...] = dinv
    g1_ref[...] = h * dinv


def _tc_stage1(deg, x, w1):
    return pl.pallas_call(
        _tc1_body,
        grid=(N // BR,),
        in_specs=[
            pl.BlockSpec((BR, D), lambda i: (i, 0)),
            pl.BlockSpec((BR, D), lambda i: (i, 0)),
            pl.BlockSpec((BR, D), lambda i: (i, 0)),
            pl.BlockSpec((D, D), lambda i: (0, 0)),
        ],
        out_specs=[
            pl.BlockSpec((BR, D), lambda i: (i, 0)),
            pl.BlockSpec((BR, D), lambda i: (i, 0)),
        ],
        out_shape=[
            jax.ShapeDtypeStruct((N, D), jnp.float32),
            jax.ShapeDtypeStruct((N, D), jnp.float32),
        ],
    )(deg[0], deg[1], x, w1)


def _tc2_body(p0_ref, p1_ref, g_ref, dinv_ref, b_ref, w_ref, out_ref):
    agg = p0_ref[...] + p1_ref[...] + g_ref[...]
    xn = jax.nn.relu(agg * dinv_ref[...] + b_ref[...])
    h = jnp.dot(xn, w_ref[...], preferred_element_type=jnp.float32)
    out_ref[...] = h * dinv_ref[...]


def _tc3_body(p0_ref, p1_ref, g_ref, dinv_ref, b_ref, w_ref, b3_ref, out_ref):
    agg = p0_ref[...] + p1_ref[...] + g_ref[...]
    xn = jax.nn.relu(agg * dinv_ref[...] + b_ref[...])
    h = jnp.dot(xn, w_ref[...], preferred_element_type=jnp.float32)
    out_ref[...] = h + b3_ref[...]


def _tc_stage2(p, g, dinv, b, w):
    return pl.pallas_call(
        _tc2_body,
        grid=(N // BR,),
        in_specs=[
            pl.BlockSpec((BR, D), lambda i: (i, 0)),
            pl.BlockSpec((BR, D), lambda i: (i, 0)),
            pl.BlockSpec((BR, D), lambda i: (i, 0)),
            pl.BlockSpec((BR, D), lambda i: (i, 0)),
            pl.BlockSpec((1, D), lambda i: (0, 0)),
            pl.BlockSpec((D, D), lambda i: (0, 0)),
        ],
        out_specs=pl.BlockSpec((BR, D), lambda i: (i, 0)),
        out_shape=jax.ShapeDtypeStruct((N, D), jnp.float32),
    )(p[0], p[1], g, dinv, b.reshape(1, D), w)


def _tc_stage3(p, g, dinv, b, w, b3):
    return pl.pallas_call(
        _tc3_body,
        grid=(N // BR,),
        in_specs=[
            pl.BlockSpec((BR, D), lambda i: (i, 0)),
            pl.BlockSpec((BR, D), lambda i: (i, 0)),
            pl.BlockSpec((BR, D), lambda i: (i, 0)),
            pl.BlockSpec((BR, D), lambda i: (i, 0)),
            pl.BlockSpec((1, D), lambda i: (0, 0)),
            pl.BlockSpec((D, D), lambda i: (0, 0)),
            pl.BlockSpec((1, D), lambda i: (0, 0)),
        ],
        out_specs=pl.BlockSpec((BR, D), lambda i: (i, 0)),
        out_shape=jax.ShapeDtypeStruct((N, D), jnp.float32),
    )(p[0], p[1], g, dinv, b.reshape(1, D), w, b3.reshape(1, D))


@jax.jit
def kernel(x, edge_index, W1, b1, W2, b2, W3, b3):
    src = edge_index[0].reshape(NW, W_CHUNKS, K)
    dst = edge_index[1].reshape(NW, W_CHUNKS, K)

    deg = _deg_kernel(dst).reshape(NC, NPAD, D)
    dinv, g1 = _tc_stage1(deg, x, W1)
    p1 = _edge_kernel(g1, src, dst).reshape(NC, NPAD, D)
    g2 = _tc_stage2(p1, g1, dinv, b1, W2)
    p2 = _edge_kernel(g2, src, dst).reshape(NC, NPAD, D)
    return _tc_stage3(p2, g2, dinv, b2, W3, b3)
